# Initial kernel scaffold; baseline (speedup 1.0000x reference)
import jax
import jax.numpy as jnp
from jax.experimental import pallas as pl


def kernel(log_probs, targets, triplets):
    raise NotImplementedError("write your pallas kernel here")


# speedup vs baseline: 5.1920x; 5.1920x over previous
import functools

import jax
import jax.numpy as jnp
from jax.experimental import pallas as pl
from jax.experimental.pallas import tpu as pltpu

LS = 0.1


def _loss_body(tgt_ref, lp_ref, out_ref, *, bn, bv):
    i = pl.program_id(0)
    j = pl.program_id(1)

    lp = jnp.maximum(lp_ref[...], -100.0)
    tgt = tgt_ref[0, 0, :]
    valid = (tgt != 1).astype(jnp.float32)

    col = j * bv + jax.lax.broadcasted_iota(jnp.int32, (bn, bv), 1)
    hit = (col == tgt[:, None]).astype(jnp.float32)

    rowsum = jnp.sum(lp, axis=1)
    tval = jnp.sum(lp * hit, axis=1)

    part_s = jnp.sum(rowsum * valid)
    part_t = jnp.sum(tval * valid)

    @pl.when((i == 0) & (j == 0))
    def _():
        out_ref[0] = 0.0
        out_ref[1] = 0.0
        out_ref[2] = 0.0

    out_ref[0] += part_s
    out_ref[1] += part_t

    @pl.when(j == 0)
    def _():
        out_ref[2] += jnp.sum(valid)


def kernel(log_probs, targets, triplets):
    n, v = log_probs.shape
    bn = 256
    bv = 3200
    nb = n // bn
    vb = v // bv

    tgt3 = targets.reshape(nb, 1, bn)

    sums = pl.pallas_call(
        functools.partial(_loss_body, bn=bn, bv=bv),
        grid=(nb, vb),
        in_specs=[
            pl.BlockSpec((1, 1, bn), lambda i, j: (i, 0, 0)),
            pl.BlockSpec((bn, bv), lambda i, j: (i, j)),
        ],
        out_specs=pl.BlockSpec(memory_space=pltpu.SMEM),
        out_shape=jax.ShapeDtypeStruct((3,), jnp.float32),
    )(tgt3, log_probs)

    s, t, c = sums[0], sums[1], sums[2]
    coef = 1.0 - LS - LS / v
    return -((LS / v) * s + coef * t) / jnp.maximum(c, 1.0)
